# baseline (speedup 1.0000x reference)
import functools

import jax
import jax.numpy as jnp
from jax import lax
from jax.experimental import pallas as pl
from jax.experimental.pallas import tpu as pltpu
from jax.experimental.pallas import tpu_sc as plsc

B = 4096
NF = 13
NC = 26
V = 100000
D = 64
H1, H2, H3 = 512, 256, 128

BNC = B * NC
NWORK = 32
PER_W = BNC // NWORK
CHUNK = 128
NCHUNK = PER_W // CHUNK

@functools.cache
def _make_sc_gather():
    mesh = plsc.VectorSubcoreMesh(core_axis_name="c", subcore_axis_name="s")

    @functools.partial(
        pl.kernel,
        mesh=mesh,
        compiler_params=pltpu.CompilerParams(use_tc_tiling_on_sc=False),
        out_type=jax.ShapeDtypeStruct((BNC, D), jnp.float32),
        scratch_types=[
            pltpu.VMEM((NCHUNK, CHUNK), jnp.int32),
            pltpu.VMEM((CHUNK, D), jnp.float32),
            pltpu.VMEM((CHUNK, D), jnp.float32),
            pltpu.SemaphoreType.DMA,
            pltpu.SemaphoreType.DMA,
        ],
    )
    def _sc_gather(idx_hbm, table_hbm, out_hbm, idx_v, rows0, rows1, sem0, sem1):
        wid = lax.axis_index("s") * 2 + lax.axis_index("c")
        base = wid * PER_W
        pltpu.sync_copy(idx_hbm.at[wid], idx_v)
        rows = (rows0, rows1)
        sems = (sem0, sem1)
        copies = [None, None]
        for j in range(NCHUNK):
            s = j % 2
            copies[s] = pltpu.async_copy(table_hbm.at[idx_v.at[j]], rows[s], sems[s])
            if j > 0:
                prev = 1 - s
                copies[prev].wait()
                pltpu.sync_copy(rows[prev],
                                out_hbm.at[pl.ds(base + (j - 1) * CHUNK, CHUNK)])
        copies[(NCHUNK - 1) % 2].wait()
        pltpu.sync_copy(rows[(NCHUNK - 1) % 2],
                        out_hbm.at[pl.ds(base + (NCHUNK - 1) * CHUNK, CHUNK)])

    return _sc_gather


BB = 512


def _mlp_body(cont_ref, emb_ref, wc_ref, bc_ref, w1a_ref, w1b_ref, b1_ref,
              w2_ref, b2_ref, w3_ref, b3_ref, wo_ref, bo_ref, out_ref):
    x = jnp.dot(cont_ref[:], wc_ref[:], preferred_element_type=jnp.float32)
    x = x + bc_ref[:]
    a = jnp.dot(x, w1a_ref[:], preferred_element_type=jnp.float32)
    a = a + jnp.dot(emb_ref[:], w1b_ref[:], preferred_element_type=jnp.float32)
    a = jnp.maximum(a + b1_ref[:], 0.0)
    a = jnp.maximum(jnp.dot(a, w2_ref[:], preferred_element_type=jnp.float32) + b2_ref[:], 0.0)
    a = jnp.maximum(jnp.dot(a, w3_ref[:], preferred_element_type=jnp.float32) + b3_ref[:], 0.0)
    o = jnp.dot(a, wo_ref[:], preferred_element_type=jnp.float32) + bo_ref[:]
    out_ref[:] = jax.nn.sigmoid(o)


def _mlp(cont, emb, W_cont, b_cont, W1a, W1b, b1, W2, b2, W3, b3, Wo, bo):
    grid = (B // BB,)
    full = lambda r, c: pl.BlockSpec((r, c), lambda i: (0, 0))
    return pl.pallas_call(
        _mlp_body,
        grid=grid,
        in_specs=[
            pl.BlockSpec((BB, NF), lambda i: (i, 0)),
            pl.BlockSpec((BB, NC * D), lambda i: (i, 0)),
            full(NF, D), full(1, D),
            full(D, H1), full(NC * D, H1), full(1, H1),
            full(H1, H2), full(1, H2),
            full(H2, H3), full(1, H3),
            full(H3, 1), full(1, 1),
        ],
        out_specs=pl.BlockSpec((BB, 1), lambda i: (i, 0)),
        out_shape=jax.ShapeDtypeStruct((B, 1), jnp.float32),
    )(cont, emb, W_cont, b_cont, W1a, W1b, b1, W2, b2, W3, b3, Wo, bo)


def kernel(continuous_features, categorical_features, tables,
           W_cont, b_cont, W1, b1, W2, b2, W3, b3, Wo, bo):
    cat = categorical_features.astype(jnp.int32)
    offs = (jnp.arange(NC, dtype=jnp.int32) * V)[None, :]
    idx = (cat + offs).reshape(NWORK, NCHUNK, CHUNK)
    flat_tables = tables.reshape(NC * V, D)

    gathered = _make_sc_gather()(idx, flat_tables)
    emb = gathered.reshape(B, NC * D)

    out = _mlp(continuous_features, emb,
               W_cont, b_cont.reshape(1, D),
               W1[:D], W1[D:], b1.reshape(1, H1),
               W2, b2.reshape(1, H2),
               W3, b3.reshape(1, H3),
               Wo, bo.reshape(1, 1))
    return out


# speedup vs baseline: 1.3853x; 1.3853x over previous
import functools

import jax
import jax.numpy as jnp
from jax import lax
from jax.experimental import pallas as pl
from jax.experimental.pallas import tpu as pltpu
from jax.experimental.pallas import tpu_sc as plsc

B = 4096
NF = 13
NC = 26
V = 100000
D = 64
H1, H2, H3 = 512, 256, 128

R = NC * D
SLABS = R // 8
NWORK = 32
SLABS_PER_W = 7
FULL_TILES = V // 128
TAIL_W = V - FULL_TILES * 128
NT = 88
_CHUNKS = [(i * NT, NT) for i in range(FULL_TILES // NT)]
_REM = FULL_TILES - (FULL_TILES // NT) * NT
if _REM:
    _CHUNKS.append((FULL_TILES - _REM, _REM))
NVEC = B // 16


@functools.cache
def _make_sc_gather():
    mesh = plsc.VectorSubcoreMesh(core_axis_name="c", subcore_axis_name="s")

    @functools.partial(
        pl.kernel,
        mesh=mesh,
        compiler_params=pltpu.CompilerParams(needs_layout_passes=False),
        out_type=jax.ShapeDtypeStruct((R, B), jnp.float32),
        scratch_types=[
            pltpu.VMEM((B,), jnp.int32),
            pltpu.VMEM((NT, 8, 128), jnp.float32),
            pltpu.VMEM((8, B), jnp.float32),
            pltpu.SemaphoreType.DMA,
        ],
    )
    def _sc_gather(cat_hbm, tbl_hbm, tail_hbm, out_hbm, idx_v, buf, outb, sem):
        wid = lax.axis_index("s") * 2 + lax.axis_index("c")

        def scan_chunk(k0, nt):
            def scan(g, c):
                iv = idx_v[pl.ds(g * 16, 16)]
                k = lax.shift_right_logical(iv, 7)
                m = (k >= k0) & (k < k0 + nt)
                kk = jnp.where(m, k - k0, 0)
                l = jnp.where(m, iv & 127, 0)
                for s in range(8):
                    sv = jnp.full((16,), s, jnp.int32)
                    got = plsc.load_gather(buf, [kk, sv, l], mask=m)
                    cur = outb[s, pl.ds(g * 16, 16)]
                    outb[s, pl.ds(g * 16, 16)] = jnp.where(m, got, cur)
                return c
            lax.fori_loop(0, NVEC, scan, 0)

        def slab_body(j, carry):
            q = wid + NWORK * j

            @pl.when(q < SLABS)
            def _():
                f = q // 8
                pltpu.sync_copy(cat_hbm.at[pl.ds(f * B, B)], idx_v)

                for (k0, nt) in _CHUNKS:
                    def issue(kk, c):
                        pltpu.async_copy(
                            tbl_hbm.at[pl.ds(q * 8, 8),
                                       pl.ds((k0 + kk) * 128, 128)],
                            buf.at[kk], sem)
                        return c
                    lax.fori_loop(0, nt, issue, 0)

                    def drain(kk, c):
                        pltpu.make_async_copy(
                            tbl_hbm.at[pl.ds(0, 8), pl.ds(0, 128)],
                            buf.at[kk], sem).wait()
                        return c
                    lax.fori_loop(0, nt, drain, 0)

                    scan_chunk(k0, nt)

                pltpu.sync_copy(tail_hbm.at[pl.ds(q * 8, 8)], buf.at[0])
                scan_chunk(FULL_TILES, 1)

                pltpu.sync_copy(outb, out_hbm.at[pl.ds(q * 8, 8)])
            return carry

        lax.fori_loop(0, SLABS_PER_W, slab_body, 0)

    return _sc_gather


BB = 512


def _mlp_body(cont_ref, emb_ref, wc_ref, bc_ref, w1a_ref, w1b_ref, b1_ref,
              w2_ref, b2_ref, w3_ref, b3_ref, wo_ref, bo_ref, out_ref):
    x = jnp.dot(cont_ref[:], wc_ref[:], preferred_element_type=jnp.float32)
    x = x + bc_ref[:]
    a = jnp.dot(x, w1a_ref[:], preferred_element_type=jnp.float32)
    a = a + lax.dot_general(emb_ref[:], w1b_ref[:],
                            dimension_numbers=(((0,), (0,)), ((), ())),
                            preferred_element_type=jnp.float32)
    a = jnp.maximum(a + b1_ref[:], 0.0)
    a = jnp.maximum(jnp.dot(a, w2_ref[:], preferred_element_type=jnp.float32) + b2_ref[:], 0.0)
    a = jnp.maximum(jnp.dot(a, w3_ref[:], preferred_element_type=jnp.float32) + b3_ref[:], 0.0)
    o = jnp.dot(a, wo_ref[:], preferred_element_type=jnp.float32) + bo_ref[:]
    out_ref[:] = jax.nn.sigmoid(o)


def _mlp(cont, embT, W_cont, b_cont, W1a, W1b, b1, W2, b2, W3, b3, Wo, bo):
    grid = (B // BB,)
    full = lambda r, c: pl.BlockSpec((r, c), lambda i: (0, 0))
    return pl.pallas_call(
        _mlp_body,
        grid=grid,
        in_specs=[
            pl.BlockSpec((BB, NF), lambda i: (i, 0)),
            pl.BlockSpec((R, BB), lambda i: (0, i)),
            full(NF, D), full(1, D),
            full(D, H1), full(R, H1), full(1, H1),
            full(H1, H2), full(1, H2),
            full(H2, H3), full(1, H3),
            full(H3, 1), full(1, 1),
        ],
        out_specs=pl.BlockSpec((BB, 1), lambda i: (i, 0)),
        out_shape=jax.ShapeDtypeStruct((B, 1), jnp.float32),
    )(cont, embT, W_cont, b_cont, W1a, W1b, b1, W2, b2, W3, b3, Wo, bo)


def kernel(continuous_features, categorical_features, tables,
           W_cont, b_cont, W1, b1, W2, b2, W3, b3, Wo, bo):
    cat = categorical_features.astype(jnp.int32)
    cat_flat = cat.T.reshape(-1)
    tbl = tables.transpose(0, 2, 1).reshape(R, V)
    tail = jnp.pad(tbl[:, FULL_TILES * 128:], ((0, 0), (0, 128 - TAIL_W)))

    embT = _make_sc_gather()(cat_flat, tbl, tail)

    out = _mlp(continuous_features, embT,
               W_cont, b_cont.reshape(1, D),
               W1[:D], W1[D:], b1.reshape(1, H1),
               W2, b2.reshape(1, H2),
               W3, b3.reshape(1, H3),
               Wo, bo.reshape(1, 1))
    return out


# speedup vs baseline: 1.5173x; 1.0953x over previous
import functools

import jax
import jax.numpy as jnp
from jax import lax
from jax.experimental import pallas as pl
from jax.experimental.pallas import tpu as pltpu
from jax.experimental.pallas import tpu_sc as plsc

B = 4096
NF = 13
NC = 26
V = 100000
D = 64
H1, H2, H3 = 512, 256, 128

R = NC * D
SLABS = R // 8
NWORK = 32
SLABS_PER_W = 7
FULL_TILES = V // 128
TAIL_W = V - FULL_TILES * 128
NT = 91
_CHUNKS = [(i * NT, NT) for i in range(FULL_TILES // NT)]
_REM = FULL_TILES - (FULL_TILES // NT) * NT
if _REM:
    _CHUNKS.append((FULL_TILES - _REM, _REM))
NVEC = B // 16


@functools.cache
def _make_sc_gather():
    mesh = plsc.VectorSubcoreMesh(core_axis_name="c", subcore_axis_name="s")

    @functools.partial(
        pl.kernel,
        mesh=mesh,
        compiler_params=pltpu.CompilerParams(needs_layout_passes=False),
        out_type=jax.ShapeDtypeStruct((R, B), jnp.float32),
        scratch_types=[
            pltpu.VMEM((B,), jnp.int32),
            pltpu.VMEM((NT, 8, 128), jnp.float32),
            pltpu.VMEM((8, B), jnp.float32),
            pltpu.SemaphoreType.DMA,
        ],
    )
    def _sc_gather(cat_hbm, tbl_hbm, tail_hbm, out_hbm, idx_v, buf, outb, sem):
        wid = lax.axis_index("s") * 2 + lax.axis_index("c")

        lane = lax.iota(jnp.int32, 16)

        def scan_chunk(k0, nt):
            def scan(g, c):
                iv = idx_v[pl.ds(g * 16, 16)]
                k = lax.shift_right_logical(iv, 7)
                m = (k >= k0) & (k < k0 + nt)
                kk = jnp.where(m, k - k0, 0)
                l = jnp.where(m, iv & 127, 0)
                bv = lane + g * 16
                for s in range(8):
                    sv = jnp.full((16,), s, jnp.int32)
                    got = plsc.load_gather(buf, [kk, sv, l], mask=m)
                    plsc.store_scatter(outb, [sv, bv], got, mask=m)
                return c
            lax.fori_loop(0, NVEC, scan, 0)

        def slab_body(j, carry):
            q = wid + NWORK * j

            @pl.when(q < SLABS)
            def _():
                f = q // 8
                pltpu.sync_copy(cat_hbm.at[pl.ds(f * B, B)], idx_v)

                for (k0, nt) in _CHUNKS:
                    def issue(kk, c):
                        pltpu.async_copy(
                            tbl_hbm.at[pl.ds(q * 8, 8),
                                       pl.ds((k0 + kk) * 128, 128)],
                            buf.at[kk], sem)
                        return c
                    lax.fori_loop(0, nt, issue, 0)

                    def drain(kk, c):
                        pltpu.make_async_copy(
                            tbl_hbm.at[pl.ds(0, 8), pl.ds(0, 128)],
                            buf.at[kk], sem).wait()
                        return c
                    lax.fori_loop(0, nt, drain, 0)

                    scan_chunk(k0, nt)

                pltpu.sync_copy(tail_hbm.at[pl.ds(q * 8, 8)], buf.at[0])
                scan_chunk(FULL_TILES, 1)

                pltpu.sync_copy(outb, out_hbm.at[pl.ds(q * 8, 8)])
            return carry

        lax.fori_loop(0, SLABS_PER_W, slab_body, 0)

    return _sc_gather


BB = 512


def _mlp_body(cont_ref, emb_ref, wc_ref, bc_ref, w1a_ref, w1b_ref, b1_ref,
              w2_ref, b2_ref, w3_ref, b3_ref, wo_ref, bo_ref, out_ref):
    x = jnp.dot(cont_ref[:], wc_ref[:], preferred_element_type=jnp.float32)
    x = x + bc_ref[:]
    a = jnp.dot(x, w1a_ref[:], preferred_element_type=jnp.float32)
    a = a + lax.dot_general(emb_ref[:], w1b_ref[:],
                            dimension_numbers=(((0,), (0,)), ((), ())),
                            preferred_element_type=jnp.float32)
    a = jnp.maximum(a + b1_ref[:], 0.0)
    a = jnp.maximum(jnp.dot(a, w2_ref[:], preferred_element_type=jnp.float32) + b2_ref[:], 0.0)
    a = jnp.maximum(jnp.dot(a, w3_ref[:], preferred_element_type=jnp.float32) + b3_ref[:], 0.0)
    o = jnp.dot(a, wo_ref[:], preferred_element_type=jnp.float32) + bo_ref[:]
    out_ref[:] = jax.nn.sigmoid(o)


def _mlp(cont, embT, W_cont, b_cont, W1a, W1b, b1, W2, b2, W3, b3, Wo, bo):
    grid = (B // BB,)
    full = lambda r, c: pl.BlockSpec((r, c), lambda i: (0, 0))
    return pl.pallas_call(
        _mlp_body,
        grid=grid,
        in_specs=[
            pl.BlockSpec((BB, NF), lambda i: (i, 0)),
            pl.BlockSpec((R, BB), lambda i: (0, i)),
            full(NF, D), full(1, D),
            full(D, H1), full(R, H1), full(1, H1),
            full(H1, H2), full(1, H2),
            full(H2, H3), full(1, H3),
            full(H3, 1), full(1, 1),
        ],
        out_specs=pl.BlockSpec((BB, 1), lambda i: (i, 0)),
        out_shape=jax.ShapeDtypeStruct((B, 1), jnp.float32),
    )(cont, embT, W_cont, b_cont, W1a, W1b, b1, W2, b2, W3, b3, Wo, bo)


def kernel(continuous_features, categorical_features, tables,
           W_cont, b_cont, W1, b1, W2, b2, W3, b3, Wo, bo):
    cat = categorical_features.astype(jnp.int32)
    cat_flat = cat.T.reshape(-1)
    tbl = tables.transpose(0, 2, 1).reshape(R, V)
    tail = jnp.pad(tbl[:, FULL_TILES * 128:], ((0, 0), (0, 128 - TAIL_W)))

    embT = _make_sc_gather()(cat_flat, tbl, tail)

    out = _mlp(continuous_features, embT,
               W_cont, b_cont.reshape(1, D),
               W1[:D], W1[D:], b1.reshape(1, H1),
               W2, b2.reshape(1, H2),
               W3, b3.reshape(1, H3),
               Wo, bo.reshape(1, 1))
    return out


# speedup vs baseline: 2.4539x; 1.6173x over previous
import functools

import jax
import jax.numpy as jnp
from jax import lax
from jax.experimental import pallas as pl
from jax.experimental.pallas import tpu as pltpu
from jax.experimental.pallas import tpu_sc as plsc

B = 4096
NF = 13
NC = 26
V = 100000
D = 64
H1, H2, H3 = 512, 256, 128

R = NC * D
SLABS = R // 8
NWORK = 32
SLABS_PER_W = 7
FULL_TILES = V // 128
TAIL_W = V - FULL_TILES * 128
NT = 91
_CHUNKS = [(i * NT, NT) for i in range(FULL_TILES // NT)]
_REM = FULL_TILES - (FULL_TILES // NT) * NT
if _REM:
    _CHUNKS.append((FULL_TILES - _REM, _REM))
NVEC = B // 16


@functools.cache
def _make_sc_gather():
    mesh = plsc.VectorSubcoreMesh(core_axis_name="c", subcore_axis_name="s")

    @functools.partial(
        pl.kernel,
        mesh=mesh,
        compiler_params=pltpu.CompilerParams(needs_layout_passes=False),
        out_type=jax.ShapeDtypeStruct((R, B), jnp.float32),
        scratch_types=[
            pltpu.VMEM((B,), jnp.int32),
            pltpu.VMEM((NT, 8, 128), jnp.float32),
            pltpu.VMEM((8, B), jnp.float32),
            pltpu.SemaphoreType.DMA,
        ],
    )
    def _sc_gather(cat_hbm, tbl_hbm, tail_hbm, out_hbm, idx_v, buf, outb, sem):
        wid = lax.axis_index("s") * 2 + lax.axis_index("c")

        lane = lax.iota(jnp.int32, 16)
        svs = [jnp.full((16,), s, jnp.int32) for s in range(8)]

        def scan_chunk(k0, nt):
            def scan(g, c):
                iv = idx_v[pl.ds(g * 16, 16)]
                k = lax.shift_right_logical(iv, 7)
                m = (k >= k0) & (k < k0 + nt)
                kk = jnp.where(m, k - k0, 0)
                l = jnp.where(m, iv & 127, 0)
                bv = lane + g * 16
                gots = [plsc.load_gather(buf, [kk, svs[s], l], mask=m)
                        for s in range(8)]
                for s in range(8):
                    plsc.store_scatter(outb, [svs[s], bv], gots[s], mask=m)
                return c
            lax.fori_loop(0, NVEC, scan, 0)

        def slab_body(j, carry):
            q = wid + NWORK * j

            @pl.when(q < SLABS)
            def _():
                f = q // 8
                pltpu.sync_copy(cat_hbm.at[pl.ds(f * B, B)], idx_v)

                for (k0, nt) in _CHUNKS:
                    def issue(kk, c):
                        pltpu.async_copy(
                            tbl_hbm.at[pl.ds(q * 8, 8),
                                       pl.ds((k0 + kk) * 128, 128)],
                            buf.at[kk], sem)
                        return c
                    lax.fori_loop(0, nt, issue, 0)

                    def drain(kk, c):
                        pltpu.make_async_copy(
                            tbl_hbm.at[pl.ds(0, 8), pl.ds(0, 128)],
                            buf.at[kk], sem).wait()
                        return c
                    lax.fori_loop(0, nt, drain, 0)

                    scan_chunk(k0, nt)

                pltpu.sync_copy(tail_hbm.at[pl.ds(q * 8, 8)], buf.at[0])
                scan_chunk(FULL_TILES, 1)

                pltpu.sync_copy(outb, out_hbm.at[pl.ds(q * 8, 8)])
            return carry

        lax.fori_loop(0, SLABS_PER_W, slab_body, 0)

    return _sc_gather


BB = 512


def _mlp_body(cont_ref, emb_ref, wc_ref, bc_ref, w1a_ref, w1b_ref, b1_ref,
              w2_ref, b2_ref, w3_ref, b3_ref, wo_ref, bo_ref, out_ref):
    x = jnp.dot(cont_ref[:], wc_ref[:], preferred_element_type=jnp.float32)
    x = x + bc_ref[:]
    a = jnp.dot(x, w1a_ref[:], preferred_element_type=jnp.float32)
    a = a + lax.dot_general(emb_ref[:], w1b_ref[:],
                            dimension_numbers=(((0,), (0,)), ((), ())),
                            preferred_element_type=jnp.float32)
    a = jnp.maximum(a + b1_ref[:], 0.0)
    a = jnp.maximum(jnp.dot(a, w2_ref[:], preferred_element_type=jnp.float32) + b2_ref[:], 0.0)
    a = jnp.maximum(jnp.dot(a, w3_ref[:], preferred_element_type=jnp.float32) + b3_ref[:], 0.0)
    o = jnp.dot(a, wo_ref[:], preferred_element_type=jnp.float32) + bo_ref[:]
    out_ref[:] = jax.nn.sigmoid(o)


def _mlp(cont, embT, W_cont, b_cont, W1a, W1b, b1, W2, b2, W3, b3, Wo, bo):
    grid = (B // BB,)
    full = lambda r, c: pl.BlockSpec((r, c), lambda i: (0, 0))
    return pl.pallas_call(
        _mlp_body,
        grid=grid,
        in_specs=[
            pl.BlockSpec((BB, NF), lambda i: (i, 0)),
            pl.BlockSpec((R, BB), lambda i: (0, i)),
            full(NF, D), full(1, D),
            full(D, H1), full(R, H1), full(1, H1),
            full(H1, H2), full(1, H2),
            full(H2, H3), full(1, H3),
            full(H3, 1), full(1, 1),
        ],
        out_specs=pl.BlockSpec((BB, 1), lambda i: (i, 0)),
        out_shape=jax.ShapeDtypeStruct((B, 1), jnp.float32),
    )(cont, embT, W_cont, b_cont, W1a, W1b, b1, W2, b2, W3, b3, Wo, bo)


def kernel(continuous_features, categorical_features, tables,
           W_cont, b_cont, W1, b1, W2, b2, W3, b3, Wo, bo):
    cat = categorical_features.astype(jnp.int32)
    cat_flat = cat.T.reshape(-1)
    tbl = tables.transpose(0, 2, 1).reshape(R, V)
    tail = jnp.pad(tbl[:, FULL_TILES * 128:], ((0, 0), (0, 128 - TAIL_W)))

    embT = _make_sc_gather()(cat_flat, tbl, tail)

    out = _mlp(continuous_features, embT,
               W_cont, b_cont.reshape(1, D),
               W1[:D], W1[D:], b1.reshape(1, H1),
               W2, b2.reshape(1, H2),
               W3, b3.reshape(1, H3),
               Wo, bo.reshape(1, 1))
    return out


# speedup vs baseline: 2.7864x; 1.1355x over previous
import functools

import jax
import jax.numpy as jnp
from jax import lax
from jax.experimental import pallas as pl
from jax.experimental.pallas import tpu as pltpu
from jax.experimental.pallas import tpu_sc as plsc

B = 4096
NF = 13
NC = 26
V = 100000
D = 64
H1, H2, H3 = 512, 256, 128

R = NC * D
SLABS = R // 8
NWORK = 32
SLABS_PER_W = 7
FULL_TILES = V // 128
TAIL_W = V - FULL_TILES * 128
NT = 91
_CHUNKS = [(i * NT, NT) for i in range(FULL_TILES // NT)]
_REM = FULL_TILES - (FULL_TILES // NT) * NT
if _REM:
    _CHUNKS.append((FULL_TILES - _REM, _REM))
NVEC = B // 16


@functools.cache
def _make_sc_gather():
    mesh = plsc.VectorSubcoreMesh(core_axis_name="c", subcore_axis_name="s")

    @functools.partial(
        pl.kernel,
        mesh=mesh,
        compiler_params=pltpu.CompilerParams(needs_layout_passes=False),
        out_type=jax.ShapeDtypeStruct((R, B), jnp.float32),
        scratch_types=[
            pltpu.VMEM((B,), jnp.int32),
            pltpu.VMEM((NT, 8, 128), jnp.float32),
            pltpu.VMEM((8, B), jnp.float32),
            pltpu.SemaphoreType.DMA,
        ],
    )
    def _sc_gather(cat_hbm, tbl_hbm, tail_hbm, out_hbm, idx_v, buf, outb, sem):
        wid = lax.axis_index("s") * 2 + lax.axis_index("c")

        lane = lax.iota(jnp.int32, 16)
        svs = [jnp.full((16,), s, jnp.int32) for s in range(8)]

        def scan_chunk(k0, nt):
            def scan(g, c):
                ms, kks, ls, bvs = [], [], [], []
                for u in range(2):
                    gg = g * 2 + u
                    iv = idx_v[pl.ds(gg * 16, 16)]
                    k = lax.shift_right_logical(iv, 7)
                    m = (k >= k0) & (k < k0 + nt)
                    ms.append(m)
                    kks.append(jnp.where(m, k - k0, 0))
                    ls.append(jnp.where(m, iv & 127, 0))
                    bvs.append(lane + gg * 16)
                gots = [[plsc.load_gather(buf, [kks[u], svs[s], ls[u]],
                                          mask=ms[u])
                         for s in range(8)] for u in range(2)]
                for u in range(2):
                    for s in range(8):
                        plsc.store_scatter(outb, [svs[s], bvs[u]], gots[u][s],
                                           mask=ms[u])
                return c
            lax.fori_loop(0, NVEC // 2, scan, 0)

        def slab_body(j, carry):
            q = wid + NWORK * j

            @pl.when(q < SLABS)
            def _():
                f = q // 8
                pltpu.sync_copy(cat_hbm.at[pl.ds(f * B, B)], idx_v)

                for (k0, nt) in _CHUNKS:
                    def issue(kk, c):
                        pltpu.async_copy(
                            tbl_hbm.at[pl.ds(q * 8, 8),
                                       pl.ds((k0 + kk) * 128, 128)],
                            buf.at[kk], sem)
                        return c
                    lax.fori_loop(0, nt, issue, 0)

                    def drain(kk, c):
                        pltpu.make_async_copy(
                            tbl_hbm.at[pl.ds(0, 8), pl.ds(0, 128)],
                            buf.at[kk], sem).wait()
                        return c
                    lax.fori_loop(0, nt, drain, 0)

                    scan_chunk(k0, nt)

                pltpu.sync_copy(tail_hbm.at[pl.ds(q * 8, 8)], buf.at[0])
                scan_chunk(FULL_TILES, 1)

                pltpu.sync_copy(outb, out_hbm.at[pl.ds(q * 8, 8)])
            return carry

        lax.fori_loop(0, SLABS_PER_W, slab_body, 0)

    return _sc_gather


BB = 512


def _mlp_body(cont_ref, emb_ref, wc_ref, bc_ref, w1a_ref, w1b_ref, b1_ref,
              w2_ref, b2_ref, w3_ref, b3_ref, wo_ref, bo_ref, out_ref):
    x = jnp.dot(cont_ref[:], wc_ref[:], preferred_element_type=jnp.float32)
    x = x + bc_ref[:]
    a = jnp.dot(x, w1a_ref[:], preferred_element_type=jnp.float32)
    a = a + lax.dot_general(emb_ref[:], w1b_ref[:],
                            dimension_numbers=(((0,), (0,)), ((), ())),
                            preferred_element_type=jnp.float32)
    a = jnp.maximum(a + b1_ref[:], 0.0)
    a = jnp.maximum(jnp.dot(a, w2_ref[:], preferred_element_type=jnp.float32) + b2_ref[:], 0.0)
    a = jnp.maximum(jnp.dot(a, w3_ref[:], preferred_element_type=jnp.float32) + b3_ref[:], 0.0)
    o = jnp.dot(a, wo_ref[:], preferred_element_type=jnp.float32) + bo_ref[:]
    out_ref[:] = jax.nn.sigmoid(o)


def _mlp(cont, embT, W_cont, b_cont, W1a, W1b, b1, W2, b2, W3, b3, Wo, bo):
    grid = (B // BB,)
    full = lambda r, c: pl.BlockSpec((r, c), lambda i: (0, 0))
    return pl.pallas_call(
        _mlp_body,
        grid=grid,
        in_specs=[
            pl.BlockSpec((BB, NF), lambda i: (i, 0)),
            pl.BlockSpec((R, BB), lambda i: (0, i)),
            full(NF, D), full(1, D),
            full(D, H1), full(R, H1), full(1, H1),
            full(H1, H2), full(1, H2),
            full(H2, H3), full(1, H3),
            full(H3, 1), full(1, 1),
        ],
        out_specs=pl.BlockSpec((BB, 1), lambda i: (i, 0)),
        out_shape=jax.ShapeDtypeStruct((B, 1), jnp.float32),
    )(cont, embT, W_cont, b_cont, W1a, W1b, b1, W2, b2, W3, b3, Wo, bo)


def kernel(continuous_features, categorical_features, tables,
           W_cont, b_cont, W1, b1, W2, b2, W3, b3, Wo, bo):
    cat = categorical_features.astype(jnp.int32)
    cat_flat = cat.T.reshape(-1)
    tbl = tables.transpose(0, 2, 1).reshape(R, V)
    tail = jnp.pad(tbl[:, FULL_TILES * 128:], ((0, 0), (0, 128 - TAIL_W)))

    embT = _make_sc_gather()(cat_flat, tbl, tail)

    out = _mlp(continuous_features, embT,
               W_cont, b_cont.reshape(1, D),
               W1[:D], W1[D:], b1.reshape(1, H1),
               W2, b2.reshape(1, H2),
               W3, b3.reshape(1, H3),
               Wo, bo.reshape(1, 1))
    return out
